# Initial kernel scaffold; baseline (speedup 1.0000x reference)
import functools
import math

import jax
import jax.numpy as jnp
from jax import lax
from jax.experimental import pallas as pl
from jax.experimental.pallas import tpu as pltpu

N = 1024
K = 128
M = N * N
TAB = 128

_SQRT_2PI = math.sqrt(2.0 * math.pi)


def _make_sc_affine():
    from jax.experimental.pallas import tpu_sc as plsc

    info = plsc.get_sparse_core_info()
    NC, NS, L = info.num_cores, info.num_subcores, info.num_lanes
    NW = NC * NS
    per_w = M // NW
    mesh = plsc.VectorSubcoreMesh(core_axis_name="c", subcore_axis_name="s")

    @functools.partial(
        pl.kernel,
        mesh=mesh,
        out_type=jax.ShapeDtypeStruct((M,), jnp.float32),
        scratch_types=[
            pltpu.VMEM((per_w,), jnp.int32),
            pltpu.VMEM((per_w,), jnp.float32),
            pltpu.VMEM((per_w,), jnp.float32),
            pltpu.VMEM((TAB,), jnp.float32),
            pltpu.VMEM((TAB,), jnp.float32),
        ],
    )
    def sc_affine(x_hbm, et_hbm, gtab_hbm, btab_hbm, out_hbm,
                  et_v, x_v, o_v, g_v, b_v):
        wid = lax.axis_index("s") * NC + lax.axis_index("c")
        base = wid * per_w
        pltpu.sync_copy(gtab_hbm, g_v)
        pltpu.sync_copy(btab_hbm, b_v)
        pltpu.sync_copy(x_hbm.at[pl.ds(base, per_w)], x_v)
        pltpu.sync_copy(et_hbm.at[pl.ds(base, per_w)], et_v)

        def body(i, _):
            sl = pl.ds(i * L, L)
            idx = et_v[sl]
            g = plsc.load_gather(g_v, [idx])
            b = plsc.load_gather(b_v, [idx])
            o_v[sl] = g * x_v[sl] + b
            return 0

        lax.fori_loop(0, per_w // L, body, 0)
        pltpu.sync_copy(o_v, out_hbm.at[pl.ds(base, per_w)])

    return sc_affine


BR = 8


def _tc_expand_body(xt_ref, m_ref, s_ref, o_ref):
    m = m_ref[...].reshape(1, 1, K)
    s = jnp.abs(s_ref[...]).reshape(1, 1, K) + 0.01
    inv = 1.0 / s
    c = inv * (1.0 / _SQRT_2PI)
    xt = xt_ref[...]
    t = (xt[:, :, None] - m) * inv
    o_ref[...] = c * jnp.exp(-0.5 * t * t)


def _tc_expand(xt, means, stds, interpret=False):
    return pl.pallas_call(
        _tc_expand_body,
        grid=(N // BR,),
        in_specs=[
            pl.BlockSpec((BR, N), lambda i: (i, 0)),
            pl.BlockSpec((1, K), lambda i: (0, 0)),
            pl.BlockSpec((1, K), lambda i: (0, 0)),
        ],
        out_specs=pl.BlockSpec((BR, N, K), lambda i: (i, 0, 0)),
        out_shape=jax.ShapeDtypeStruct((N, N, K), jnp.float32),
        interpret=interpret,
    )(xt, means, stds)


def kernel(x, edge_types, means, stds, gamma, beta):
    gtab = jnp.pad(gamma[:, 0], (0, TAB - gamma.shape[0]))
    btab = jnp.pad(beta[:, 0], (0, TAB - beta.shape[0]))
    sc_affine = _make_sc_affine()
    xt = sc_affine(x.reshape(M), edge_types.reshape(M).astype(jnp.int32),
                   gtab, btab)
    return _tc_expand(xt.reshape(N, N), means, stds)


# speedup vs baseline: 105.0279x; 105.0279x over previous
import functools
import math

import jax
import jax.numpy as jnp
from jax import lax
from jax.experimental import pallas as pl
from jax.experimental.pallas import tpu as pltpu

N = 1024
K = 128
M = N * N
TAB = 128
CH = 4
BR = 32

_SQRT_2PI = math.sqrt(2.0 * math.pi)


def _make_sc_affine(m_chunk):
    from jax.experimental.pallas import tpu_sc as plsc

    info = plsc.get_sparse_core_info()
    NC, NS, L = info.num_cores, info.num_subcores, info.num_lanes
    NW = NC * NS
    per_w = m_chunk // NW
    mesh = plsc.VectorSubcoreMesh(core_axis_name="c", subcore_axis_name="s")

    @functools.partial(
        pl.kernel,
        mesh=mesh,
        out_type=jax.ShapeDtypeStruct((m_chunk,), jnp.float32),
        compiler_params=pltpu.CompilerParams(needs_layout_passes=False),
        scratch_types=[
            pltpu.VMEM((per_w,), jnp.int32),
            pltpu.VMEM((per_w,), jnp.float32),
            pltpu.VMEM((per_w,), jnp.float32),
            pltpu.VMEM((TAB,), jnp.float32),
            pltpu.VMEM((TAB,), jnp.float32),
        ],
    )
    def sc_affine(x_hbm, et_hbm, gtab_hbm, btab_hbm, out_hbm,
                  et_v, x_v, o_v, g_v, b_v):
        wid = lax.axis_index("s") * NC + lax.axis_index("c")
        base = wid * per_w
        pltpu.sync_copy(gtab_hbm, g_v)
        pltpu.sync_copy(btab_hbm, b_v)
        pltpu.sync_copy(x_hbm.at[pl.ds(base, per_w)], x_v)
        pltpu.sync_copy(et_hbm.at[pl.ds(base, per_w)], et_v)

        def body(i, _):
            sl = pl.ds(i * L, L)
            idx = et_v[sl]
            g = plsc.load_gather(g_v, [idx])
            b = plsc.load_gather(b_v, [idx])
            o_v[sl] = g * x_v[sl] + b
            return 0

        lax.fori_loop(0, per_w // L, body, 0)
        pltpu.sync_copy(o_v, out_hbm.at[pl.ds(base, per_w)])

    return sc_affine


def _tc_body_first(xt_ref, m_ref, s_ref, o_ref):
    _tc_compute(xt_ref, m_ref, s_ref, o_ref)


def _tc_body_chained(xt_ref, m_ref, s_ref, psi_ref, o_ref):
    del psi_ref
    _tc_compute(xt_ref, m_ref, s_ref, o_ref)


def _tc_compute(xt_ref, m_ref, s_ref, o_ref):
    m = m_ref[...].reshape(1, 1, K)
    s = jnp.abs(s_ref[...]).reshape(1, 1, K) + 0.01
    inv = 1.0 / s
    c = inv * (1.0 / _SQRT_2PI)
    b = inv * inv * (-0.5 * math.log2(math.e))
    xt = xt_ref[...]
    d = xt[:, :, None] - m
    o_ref[...] = c * jnp.exp2(d * d * b)


def _tc_expand_chunk(xt_c, means, stds, psi, row0):
    rows = xt_c.shape[0]
    nblk = rows // BR
    blk0 = row0 // BR
    xt_spec = pl.BlockSpec((BR, N), lambda i: (i, 0))
    k_spec = pl.BlockSpec((1, K), lambda i: (0, 0))
    out_spec = pl.BlockSpec((BR, N, K), lambda i: (i + blk0, 0, 0))
    out_shape = jax.ShapeDtypeStruct((N, N, K), jnp.float32)
    if psi is None:
        return pl.pallas_call(
            _tc_body_first,
            grid=(nblk,),
            in_specs=[xt_spec, k_spec, k_spec],
            out_specs=out_spec,
            out_shape=out_shape,
        )(xt_c, means, stds)
    return pl.pallas_call(
        _tc_body_chained,
        grid=(nblk,),
        in_specs=[xt_spec, k_spec, k_spec,
                  pl.BlockSpec(memory_space=pl.ANY)],
        out_specs=out_spec,
        out_shape=out_shape,
        input_output_aliases={3: 0},
    )(xt_c, means, stds, psi)


def kernel(x, edge_types, means, stds, gamma, beta):
    gtab = jnp.pad(gamma[:, 0], (0, TAB - gamma.shape[0]))
    btab = jnp.pad(beta[:, 0], (0, TAB - beta.shape[0]))
    x_flat = x.reshape(M)
    et_flat = edge_types.reshape(M).astype(jnp.int32)

    m_chunk = M // CH
    rows_chunk = N // CH
    sc_affine = _make_sc_affine(m_chunk)
    xts = [
        sc_affine(
            lax.dynamic_slice(x_flat, (c * m_chunk,), (m_chunk,)),
            lax.dynamic_slice(et_flat, (c * m_chunk,), (m_chunk,)),
            gtab, btab,
        )
        for c in range(CH)
    ]

    psi = None
    for c in range(CH):
        psi = _tc_expand_chunk(xts[c].reshape(rows_chunk, N), means, stds,
                               psi, c * rows_chunk)
    return psi


# speedup vs baseline: 116.3280x; 1.1076x over previous
import functools
import math

import jax
import jax.numpy as jnp
from jax import lax
from jax.experimental import pallas as pl
from jax.experimental.pallas import tpu as pltpu

N = 1024
K = 128
M = N * N
TAB = 128
BR = 32

_SQRT_2PI = math.sqrt(2.0 * math.pi)


def _make_sc_affine():
    from jax.experimental.pallas import tpu_sc as plsc

    info = plsc.get_sparse_core_info()
    NC, NS, L = info.num_cores, info.num_subcores, info.num_lanes
    NW = NC * NS
    per_w = M // NW
    mesh = plsc.VectorSubcoreMesh(core_axis_name="c", subcore_axis_name="s")

    NCHUNK = 4
    per_c = per_w // NCHUNK

    @functools.partial(
        pl.kernel,
        mesh=mesh,
        out_type=jax.ShapeDtypeStruct((M,), jnp.float32),
        compiler_params=pltpu.CompilerParams(needs_layout_passes=False),
        scratch_types=[
            pltpu.VMEM((per_w,), jnp.int32),
            pltpu.VMEM((per_w,), jnp.float32),
            pltpu.VMEM((per_w,), jnp.float32),
            pltpu.VMEM((TAB,), jnp.float32),
            pltpu.VMEM((TAB,), jnp.float32),
            pltpu.SemaphoreType.DMA,
            pltpu.SemaphoreType.DMA,
            pltpu.SemaphoreType.DMA,
        ],
    )
    def sc_affine(x_hbm, et_hbm, gtab_hbm, btab_hbm, out_hbm,
                  et_v, x_v, o_v, g_v, b_v, sem_x, sem_e, sem_o):
        wid = lax.axis_index("s") * NC + lax.axis_index("c")
        base = wid * per_w
        in_x = []
        in_e = []
        for ci in range(NCHUNK):
            sl_v = pl.ds(ci * per_c, per_c)
            sl_h = pl.ds(base + ci * per_c, per_c)
            in_x.append(pltpu.async_copy(x_hbm.at[sl_h], x_v.at[sl_v], sem_x))
            in_e.append(pltpu.async_copy(et_hbm.at[sl_h], et_v.at[sl_v],
                                         sem_e))
        pltpu.sync_copy(gtab_hbm, g_v)
        pltpu.sync_copy(btab_hbm, b_v)

        out_cp = []
        for ci in range(NCHUNK):
            in_x[ci].wait()
            in_e[ci].wait()

            @plsc.parallel_loop(ci * (per_c // L), (ci + 1) * (per_c // L),
                                1, unroll=8)
            def body(i):
                sl = pl.ds(i * L, L)
                idx = et_v[sl]
                g = plsc.load_gather(g_v, [idx])
                b = plsc.load_gather(b_v, [idx])
                o_v[sl] = g * x_v[sl] + b

            sl_v = pl.ds(ci * per_c, per_c)
            sl_h = pl.ds(base + ci * per_c, per_c)
            out_cp.append(pltpu.async_copy(o_v.at[sl_v], out_hbm.at[sl_h],
                                           sem_o))
        for cp in out_cp:
            cp.wait()

    return sc_affine


def _tc_expand_body(xt_ref, m_ref, s_ref, o_ref):
    m = m_ref[...].reshape(1, 1, K)
    s = jnp.abs(s_ref[...]).reshape(1, 1, K) + 0.01
    inv = 1.0 / s
    c = inv * (1.0 / _SQRT_2PI)
    b = inv * inv * (-0.5 * math.log2(math.e))
    xt = xt_ref[...]
    d = xt[:, :, None] - m
    o_ref[...] = c * jnp.exp2(d * d * b)


def _tc_expand(xt, means, stds, interpret=False):
    return pl.pallas_call(
        _tc_expand_body,
        grid=(N // BR,),
        in_specs=[
            pl.BlockSpec((BR, N), lambda i: (i, 0)),
            pl.BlockSpec((1, K), lambda i: (0, 0)),
            pl.BlockSpec((1, K), lambda i: (0, 0)),
        ],
        out_specs=pl.BlockSpec((BR, N, K), lambda i: (i, 0, 0)),
        out_shape=jax.ShapeDtypeStruct((N, N, K), jnp.float32),
        interpret=interpret,
    )(xt, means, stds)


def kernel(x, edge_types, means, stds, gamma, beta):
    gtab = jnp.pad(gamma[:, 0], (0, TAB - gamma.shape[0]))
    btab = jnp.pad(beta[:, 0], (0, TAB - beta.shape[0]))
    sc_affine = _make_sc_affine()
    xt = sc_affine(x.reshape(M), edge_types.reshape(M).astype(jnp.int32),
                   gtab, btab)
    return _tc_expand(xt.reshape(N, N), means, stds)


# speedup vs baseline: 120.7881x; 1.0383x over previous
import functools
import math

import jax
import jax.numpy as jnp
from jax import lax
from jax.experimental import pallas as pl
from jax.experimental.pallas import tpu as pltpu

N = 1024
K = 128
TAB = 128
BR = 32

_SQRT_2PI = math.sqrt(2.0 * math.pi)


def _make_sc_affine():
    from jax.experimental.pallas import tpu_sc as plsc

    info = plsc.get_sparse_core_info()
    NC, NS, L = info.num_cores, info.num_subcores, info.num_lanes
    NW = NC * NS
    rows_w = N // NW
    RCH = 8
    nchunk = rows_w // RCH
    mesh = plsc.VectorSubcoreMesh(core_axis_name="c", subcore_axis_name="s")

    @functools.partial(
        pl.kernel,
        mesh=mesh,
        out_type=jax.ShapeDtypeStruct((N, N), jnp.float32),
        compiler_params=pltpu.CompilerParams(needs_layout_passes=False),
        scratch_types=[
            pltpu.VMEM((rows_w, N), jnp.int32),
            pltpu.VMEM((rows_w, N), jnp.float32),
            pltpu.VMEM((rows_w, N), jnp.float32),
            pltpu.VMEM((TAB,), jnp.float32),
            pltpu.VMEM((TAB,), jnp.float32),
            pltpu.SemaphoreType.DMA,
            pltpu.SemaphoreType.DMA,
            pltpu.SemaphoreType.DMA,
        ],
    )
    def sc_affine(x_hbm, et_hbm, gtab_hbm, btab_hbm, out_hbm,
                  et_v, x_v, o_v, g_v, b_v, sem_x, sem_e, sem_o):
        wid = lax.axis_index("s") * NC + lax.axis_index("c")
        r0 = wid * rows_w
        in_x = []
        in_e = []
        for ci in range(nchunk):
            sl_h = pl.ds(r0 + ci * RCH, RCH)
            sl_v = pl.ds(ci * RCH, RCH)
            in_x.append(pltpu.async_copy(x_hbm.at[sl_h], x_v.at[sl_v],
                                         sem_x))
            in_e.append(pltpu.async_copy(et_hbm.at[sl_h], et_v.at[sl_v],
                                         sem_e))
        pltpu.sync_copy(gtab_hbm, g_v)
        pltpu.sync_copy(btab_hbm, b_v)

        out_cp = []
        for ci in range(nchunk):
            in_x[ci].wait()
            in_e[ci].wait()
            for r in range(RCH):
                row = ci * RCH + r

                @plsc.parallel_loop(0, N // L, unroll=8)
                def body(i, row=row):
                    sl = pl.ds(i * L, L)
                    idx = et_v[row, sl]
                    g = plsc.load_gather(g_v, [idx])
                    b = plsc.load_gather(b_v, [idx])
                    o_v[row, sl] = g * x_v[row, sl] + b

            sl_h = pl.ds(r0 + ci * RCH, RCH)
            sl_v = pl.ds(ci * RCH, RCH)
            out_cp.append(pltpu.async_copy(o_v.at[sl_v], out_hbm.at[sl_h],
                                           sem_o))
        for cp in out_cp:
            cp.wait()

    return sc_affine


def _tc_expand_body(xt_ref, m_ref, s_ref, o_ref):
    m = m_ref[...].reshape(1, 1, K)
    s = jnp.abs(s_ref[...]).reshape(1, 1, K) + 0.01
    inv = 1.0 / s
    c = inv * (1.0 / _SQRT_2PI)
    b = inv * inv * (-0.5 * math.log2(math.e))
    xt = xt_ref[...]
    d = xt[:, :, None] - m
    o_ref[...] = c * jnp.exp2(d * d * b)


def _tc_expand(xt, means, stds, interpret=False):
    return pl.pallas_call(
        _tc_expand_body,
        grid=(N // BR,),
        in_specs=[
            pl.BlockSpec((BR, N), lambda i: (i, 0)),
            pl.BlockSpec((1, K), lambda i: (0, 0)),
            pl.BlockSpec((1, K), lambda i: (0, 0)),
        ],
        out_specs=pl.BlockSpec((BR, N, K), lambda i: (i, 0, 0)),
        out_shape=jax.ShapeDtypeStruct((N, N, K), jnp.float32),
        interpret=interpret,
    )(xt, means, stds)


def kernel(x, edge_types, means, stds, gamma, beta):
    gtab = jnp.pad(gamma[:, 0], (0, TAB - gamma.shape[0]))
    btab = jnp.pad(beta[:, 0], (0, TAB - beta.shape[0]))
    sc_affine = _make_sc_affine()
    xt = sc_affine(x, edge_types.astype(jnp.int32), gtab, btab)
    return _tc_expand(xt, means, stds)
